# baseline (speedup 1.0000x reference)
import functools

import jax
import jax.numpy as jnp
from jax import lax
from jax.experimental import pallas as pl
from jax.experimental.pallas import tpu as pltpu
from jax.experimental.pallas import tpu_sc as plsc

EMBED_DIM = 64
BATCH = 16384
VOCAB = 14641

_NC, _NS = 2, 16
_NW = _NC * _NS
_NG = EMBED_DIM // 8
_NQ = _NW // _NG
_BPQ = BATCH // _NQ
_L = 16


def _make_gather():
    mesh = plsc.VectorSubcoreMesh(core_axis_name="c", subcore_axis_name="s")

    @functools.partial(
        pl.kernel,
        mesh=mesh,
        out_type=jax.ShapeDtypeStruct((EMBED_DIM, BATCH), jnp.float32),
        scratch_types=[
            pltpu.VMEM((8, VOCAB), jnp.float32),
            pltpu.VMEM((_BPQ,), jnp.int32),
            pltpu.VMEM((2, 1, _BPQ), jnp.float32),
            pltpu.SemaphoreType.DMA,
            pltpu.SemaphoreType.DMA((8,)),
            pltpu.SemaphoreType.DMA,
        ],
        compiler_params=pltpu.CompilerParams(
            use_tc_tiling_on_sc=True,
            needs_layout_passes=False,
            disable_bounds_checks=True,
            disable_semaphore_checks=True,
            skip_device_barrier=True,
        ),
    )
    def gather_kernel(tabT_hbm, idx_hbm, outT_hbm, tab_v, idx_v, ob_v,
                      isem, lsems, ssem):
        wid = lax.axis_index("s") * _NC + lax.axis_index("c")
        g = wid % _NG
        q = wid // _NG

        load_idx = pltpu.async_copy(
            idx_hbm.at[pl.ds(q * _BPQ, _BPQ)], idx_v, isem)
        rows = [lax.rem(p + 2 * q, 8) for p in range(8)]
        row_loads = [
            pltpu.async_copy(
                tabT_hbm.at[pl.ds(g * 8 + rows[p], 1), :],
                tab_v.at[pl.ds(rows[p], 1)],
                lsems.at[p],
            )
            for p in range(8)
        ]
        load_idx.wait()

        def _store_desc(b):
            return pltpu.make_async_copy(
                ob_v.at[b],
                outT_hbm.at[pl.ds(g * 8 + b, 1), pl.ds(q * _BPQ, _BPQ)],
                ssem,
            )

        for p in range(8):
            b = p % 2
            r = rows[p]
            row_id = jnp.full((_L,), r, dtype=jnp.int32)
            row_loads[p].wait()
            if p >= 2:
                _store_desc(b).wait()

            @plsc.parallel_loop(0, _BPQ // _L, 1, unroll=16)
            def _group(t):
                col_idx = idx_v[pl.ds(t * _L, _L)]
                vals = plsc.load_gather(tab_v, [row_id, col_idx])
                ob_v[b, 0, pl.ds(t * _L, _L)] = vals

            pltpu.async_copy(
                ob_v.at[b],
                outT_hbm.at[pl.ds(g * 8 + r, 1), pl.ds(q * _BPQ, _BPQ)],
                ssem,
            )
        for b in range(2):
            _store_desc(b).wait()

    return gather_kernel


_gather = _make_gather()


def kernel(table, indices):
    return _gather(table.T, indices).T


# speedup vs baseline: 1.0437x; 1.0437x over previous
import functools

import jax
import jax.numpy as jnp
from jax import lax
from jax.experimental import pallas as pl
from jax.experimental.pallas import tpu as pltpu
from jax.experimental.pallas import tpu_sc as plsc

EMBED_DIM = 64
BATCH = 16384
VOCAB = 14641

_NC, _NS = 2, 16
_NW = _NC * _NS
_NG = EMBED_DIM // 8
_NQ = _NW // _NG
_BPQ = BATCH // _NQ
_L = 16


def _make_gather():
    mesh = plsc.VectorSubcoreMesh(core_axis_name="c", subcore_axis_name="s")

    @functools.partial(
        pl.kernel,
        mesh=mesh,
        out_type=jax.ShapeDtypeStruct((EMBED_DIM, BATCH), jnp.float32),
        scratch_types=[
            pltpu.VMEM((8, VOCAB), jnp.float32),
            pltpu.VMEM((_BPQ,), jnp.int32),
            pltpu.VMEM((2, 1, _BPQ), jnp.float32),
            pltpu.SemaphoreType.DMA,
            pltpu.SemaphoreType.DMA((8,)),
            pltpu.SemaphoreType.DMA,
        ],
        compiler_params=pltpu.CompilerParams(
            use_tc_tiling_on_sc=True,
            needs_layout_passes=False,
            disable_bounds_checks=True,
            disable_semaphore_checks=True,
            skip_device_barrier=True,
        ),
    )
    def gather_kernel(tabT_hbm, idx_hbm, outT_hbm, tab_v, idx_v, ob_v,
                      isem, lsems, ssem):
        wid = lax.axis_index("s") * _NC + lax.axis_index("c")
        g = wid % _NG
        q = wid // _NG

        load_idx = pltpu.async_copy(
            idx_hbm.at[pl.ds(q * _BPQ, _BPQ)], idx_v, isem)
        rows = [lax.rem(p + 2 * q, 8) for p in range(8)]
        row_loads = [
            pltpu.async_copy(
                tabT_hbm.at[pl.ds(g * 8 + rows[p], 1), :],
                tab_v.at[pl.ds(rows[p], 1)],
                lsems.at[p],
            )
            for p in range(8)
        ]
        load_idx.wait()

        def _store_desc(b):
            return pltpu.make_async_copy(
                ob_v.at[b],
                outT_hbm.at[pl.ds(g * 8 + b, 1), pl.ds(q * _BPQ, _BPQ)],
                ssem,
            )

        for p in range(8):
            b = p % 2
            r = rows[p]
            row_id = jnp.full((_L,), r, dtype=jnp.int32)
            row_loads[p].wait()
            if p >= 2:
                _store_desc(b).wait()

            @plsc.parallel_loop(0, _BPQ // _L, 1, unroll=8)
            def _group(t):
                col_idx = idx_v[pl.ds(t * _L, _L)]
                vals = plsc.load_gather(tab_v, [row_id, col_idx])
                ob_v[b, 0, pl.ds(t * _L, _L)] = vals

            pltpu.async_copy(
                ob_v.at[b],
                outT_hbm.at[pl.ds(g * 8 + r, 1), pl.ds(q * _BPQ, _BPQ)],
                ssem,
            )
        for b in range(2):
            _store_desc(b).wait()

    return gather_kernel


_gather = _make_gather()


def kernel(table, indices):
    return _gather(table.T, indices).T
